# Initial kernel scaffold; baseline (speedup 1.0000x reference)
import jax
import jax.numpy as jnp
from jax.experimental import pallas as pl


def kernel(edge_attr, edge_index, proj_W, proj_b, upd_W, upd_b):
    raise NotImplementedError("write your pallas kernel here")


# speedup vs baseline: 7.2977x; 7.2977x over previous
import functools

import jax
import jax.numpy as jnp
from jax import lax
from jax.experimental import pallas as pl
from jax.experimental.pallas import tpu as pltpu
from jax.experimental.pallas import tpu_sc as plsc

NUM_NODES_IN = 10000
NUM_EDGES_IN = 320000
EDGE_DIM = 16
HIDDEN_DIM = 128

NC = 2
NS = 16
NW = NC * NS

N_PAD = 10240
ROWS_PER_TILE = N_PAD // NS
EPW = NUM_EDGES_IN // NW
SUB = 125
SPW = EPW // SUB
STAGE = 2000
SUBS_PER_STAGE = STAGE // SUB
N_STAGES = EPW // STAGE


def _sc_segment_sum(edge_attr, dst2d):
    mesh = plsc.VectorSubcoreMesh(core_axis_name="c", subcore_axis_name="s")

    @functools.partial(
        pl.kernel,
        mesh=mesh,
        compiler_params=pltpu.CompilerParams(use_tc_tiling_on_sc=False),
        out_type=(
            jax.ShapeDtypeStruct((NC, N_PAD, EDGE_DIM), jnp.float32),
            jax.ShapeDtypeStruct((NC * N_PAD,), jnp.float32),
        ),
        scratch_types=[
            pltpu.VMEM((STAGE, EDGE_DIM), jnp.float32),
            pltpu.VMEM((SPW, SUB), jnp.int32),
            pltpu.VMEM((128,), jnp.float32),
            pltpu.VMEM((ROWS_PER_TILE, EDGE_DIM), jnp.float32),
            pltpu.VMEM((ROWS_PER_TILE,), jnp.float32),
            pltpu.VMEM_SHARED((N_PAD, EDGE_DIM), jnp.float32),
            pltpu.VMEM_SHARED((N_PAD,), jnp.float32),
        ],
    )
    def seg(edge_hbm, dst_hbm, part_out, cnt_out,
            buf, idxv, onesv, zbuf, zcnt, acc_sh, cnt_sh):
        cid = lax.axis_index("c")
        sid = lax.axis_index("s")
        wid = cid * NS + sid

        ones16 = jnp.full((16,), 1.0, dtype=jnp.float32)
        zeros16 = jnp.zeros((16,), dtype=jnp.float32)

        for i in range(8):
            onesv[pl.ds(i * 16, 16)] = ones16

        def zrow(i, _):
            zbuf[i, :] = zeros16
            return 0
        lax.fori_loop(0, ROWS_PER_TILE, zrow, 0)

        def zrow1(i, _):
            zcnt[pl.ds(i * 16, 16)] = zeros16
            return 0
        lax.fori_loop(0, ROWS_PER_TILE // 16, zrow1, 0)

        row0 = sid * ROWS_PER_TILE
        pltpu.sync_copy(zbuf, acc_sh.at[pl.ds(row0, ROWS_PER_TILE)])
        pltpu.sync_copy(zcnt, cnt_sh.at[pl.ds(row0, ROWS_PER_TILE)])
        plsc.subcore_barrier()

        pltpu.sync_copy(dst_hbm.at[pl.ds(wid * SPW, SPW)], idxv)

        ebase = wid * EPW

        def stage_body(g, _):
            pltpu.sync_copy(edge_hbm.at[pl.ds(ebase + g * STAGE, STAGE)], buf)

            def sub_body(t, _):
                j = g * SUBS_PER_STAGE + t
                idx = idxv.at[j]
                pltpu.sync_copy(buf.at[pl.ds(t * SUB, SUB)],
                                acc_sh.at[idx], add=True)
                pltpu.sync_copy(onesv.at[pl.ds(0, SUB)], cnt_sh.at[idx],
                                add=True)
                return 0
            lax.fori_loop(0, SUBS_PER_STAGE, sub_body, 0)
            return 0
        lax.fori_loop(0, N_STAGES, stage_body, 0)

        plsc.subcore_barrier()

        pltpu.sync_copy(acc_sh.at[pl.ds(row0, ROWS_PER_TILE)], zbuf)
        pltpu.sync_copy(zbuf, part_out.at[cid, pl.ds(row0, ROWS_PER_TILE)])
        pltpu.sync_copy(cnt_sh.at[pl.ds(row0, ROWS_PER_TILE)], zcnt)
        pltpu.sync_copy(
            zcnt, cnt_out.at[pl.ds(cid * N_PAD + row0, ROWS_PER_TILE)])

    return seg(edge_attr, dst2d)


def _tc_dense_body(p_ref, c_ref, w1_ref, b1_ref, w2_ref, b2_ref, o_ref):
    s = p_ref[0] + p_ref[1]
    n = c_ref[:, 0:1] + c_ref[:, 1:2]
    c = jnp.maximum(n, 1.0)
    m = jnp.dot(s, w1_ref[...], preferred_element_type=jnp.float32)
    agg = (m + n * b1_ref[...]) / c
    o_ref[...] = (
        jnp.dot(agg, w2_ref[...], preferred_element_type=jnp.float32)
        + b2_ref[...]
    )


def _tc_dense(part, cnt_t, proj_W, proj_b2, upd_W, upd_b2):
    R = 1024
    grid = (N_PAD // R,)
    return pl.pallas_call(
        _tc_dense_body,
        grid=grid,
        in_specs=[
            pl.BlockSpec((NC, R, EDGE_DIM), lambda i: (0, i, 0)),
            pl.BlockSpec((R, NC), lambda i: (i, 0)),
            pl.BlockSpec((EDGE_DIM, HIDDEN_DIM), lambda i: (0, 0)),
            pl.BlockSpec((1, HIDDEN_DIM), lambda i: (0, 0)),
            pl.BlockSpec((HIDDEN_DIM, HIDDEN_DIM), lambda i: (0, 0)),
            pl.BlockSpec((1, HIDDEN_DIM), lambda i: (0, 0)),
        ],
        out_specs=pl.BlockSpec((R, HIDDEN_DIM), lambda i: (i, 0)),
        out_shape=jax.ShapeDtypeStruct((N_PAD, HIDDEN_DIM), jnp.float32),
    )(part, cnt_t, proj_W, proj_b2, upd_W, upd_b2)


@jax.jit
def kernel(edge_attr, edge_index, proj_W, proj_b, upd_W, upd_b):
    dst = edge_index[1].astype(jnp.int32)
    dst2d = dst.reshape(NUM_EDGES_IN // SUB, SUB)
    part, cnt = _sc_segment_sum(edge_attr, dst2d)
    out = _tc_dense(
        part,
        cnt.reshape(NC, N_PAD).T,
        proj_W,
        proj_b.reshape(1, HIDDEN_DIM),
        upd_W,
        upd_b.reshape(1, HIDDEN_DIM),
    )
    return out[:NUM_NODES_IN]


# speedup vs baseline: 7.3962x; 1.0135x over previous
import functools

import jax
import jax.numpy as jnp
from jax import lax
from jax.experimental import pallas as pl
from jax.experimental.pallas import tpu as pltpu
from jax.experimental.pallas import tpu_sc as plsc

NUM_NODES_IN = 10000
NUM_EDGES_IN = 320000
EDGE_DIM = 16
HIDDEN_DIM = 128

NC = 2
NS = 16
NW = NC * NS

N_PAD = 10240
ROWS_PER_TILE = N_PAD // NS
SUB = 128
N_ROWS = NUM_EDGES_IN // SUB
ROWS_PER_W = N_ROWS // NW
EXTRA_W = N_ROWS % NW
CHUNK_ROWS = 13
N_CHUNKS = ROWS_PER_W // CHUNK_ROWS


def _sc_segment_sum(edge_attr, dst2d):
    mesh = plsc.VectorSubcoreMesh(core_axis_name="c", subcore_axis_name="s")

    @functools.partial(
        pl.kernel,
        mesh=mesh,
        compiler_params=pltpu.CompilerParams(use_tc_tiling_on_sc=False),
        out_type=(
            jax.ShapeDtypeStruct((NC, N_PAD, EDGE_DIM), jnp.float32),
            jax.ShapeDtypeStruct((NC * N_PAD,), jnp.float32),
        ),
        scratch_types=[
            pltpu.VMEM((CHUNK_ROWS * SUB, EDGE_DIM), jnp.float32),
            pltpu.VMEM((ROWS_PER_W, SUB), jnp.int32),
            pltpu.VMEM((1, SUB), jnp.int32),
            pltpu.VMEM((SUB,), jnp.float32),
            pltpu.VMEM((ROWS_PER_TILE, EDGE_DIM), jnp.float32),
            pltpu.VMEM((ROWS_PER_TILE,), jnp.float32),
            pltpu.VMEM_SHARED((N_PAD, EDGE_DIM), jnp.float32),
            pltpu.VMEM_SHARED((N_PAD,), jnp.float32),
        ],
    )
    def seg(edge_hbm, dst_hbm, part_out, cnt_out,
            buf, idxv, idxx, onesv, zbuf, zcnt, acc_sh, cnt_sh):
        cid = lax.axis_index("c")
        sid = lax.axis_index("s")
        wid = cid * NS + sid
        base_row = wid * ROWS_PER_W + jnp.minimum(wid, EXTRA_W)
        ebase = base_row * SUB

        ones16 = jnp.full((16,), 1.0, dtype=jnp.float32)
        zeros16 = jnp.zeros((16,), dtype=jnp.float32)

        for i in range(SUB // 16):
            onesv[pl.ds(i * 16, 16)] = ones16

        def zrow(i, _):
            zbuf[i, :] = zeros16
            return 0
        lax.fori_loop(0, ROWS_PER_TILE, zrow, 0)

        def zrow1(i, _):
            zcnt[pl.ds(i * 16, 16)] = zeros16
            return 0
        lax.fori_loop(0, ROWS_PER_TILE // 16, zrow1, 0)

        row0 = sid * ROWS_PER_TILE
        pltpu.sync_copy(zbuf, acc_sh.at[pl.ds(row0, ROWS_PER_TILE)])
        pltpu.sync_copy(zcnt, cnt_sh.at[pl.ds(row0, ROWS_PER_TILE)])
        plsc.subcore_barrier()

        pltpu.sync_copy(dst_hbm.at[pl.ds(base_row, ROWS_PER_W)], idxv)

        def chunk_body(g, _):
            pltpu.sync_copy(
                edge_hbm.at[pl.ds(ebase + g * (CHUNK_ROWS * SUB),
                                  CHUNK_ROWS * SUB)], buf)

            def sub_body(t, _):
                idx = idxv.at[g * CHUNK_ROWS + t]
                pltpu.sync_copy(buf.at[pl.ds(t * SUB, SUB)],
                                acc_sh.at[idx], add=True)
                pltpu.sync_copy(onesv, cnt_sh.at[idx], add=True)
                return 0
            lax.fori_loop(0, CHUNK_ROWS, sub_body, 0)
            return 0
        lax.fori_loop(0, N_CHUNKS, chunk_body, 0)

        @pl.when(wid < EXTRA_W)
        def _():
            xrow = base_row + ROWS_PER_W
            pltpu.sync_copy(dst_hbm.at[pl.ds(xrow, 1)], idxx)
            pltpu.sync_copy(edge_hbm.at[pl.ds(xrow * SUB, SUB)],
                            buf.at[pl.ds(0, SUB)])
            idx = idxx.at[0]
            pltpu.sync_copy(buf.at[pl.ds(0, SUB)], acc_sh.at[idx], add=True)
            pltpu.sync_copy(onesv, cnt_sh.at[idx], add=True)

        plsc.subcore_barrier()

        pltpu.sync_copy(acc_sh.at[pl.ds(row0, ROWS_PER_TILE)], zbuf)
        pltpu.sync_copy(zbuf, part_out.at[cid, pl.ds(row0, ROWS_PER_TILE)])
        pltpu.sync_copy(cnt_sh.at[pl.ds(row0, ROWS_PER_TILE)], zcnt)
        pltpu.sync_copy(
            zcnt, cnt_out.at[pl.ds(cid * N_PAD + row0, ROWS_PER_TILE)])

    return seg(edge_attr, dst2d)


def _tc_dense_body(p_ref, c_ref, w1_ref, b1_ref, w2_ref, b2_ref, o_ref):
    s = p_ref[0] + p_ref[1]
    n = c_ref[:, 0:1] + c_ref[:, 1:2]
    c = jnp.maximum(n, 1.0)
    m = jnp.dot(s, w1_ref[...], preferred_element_type=jnp.float32)
    agg = (m + n * b1_ref[...]) / c
    o_ref[...] = (
        jnp.dot(agg, w2_ref[...], preferred_element_type=jnp.float32)
        + b2_ref[...]
    )


def _tc_dense(part, cnt_t, proj_W, proj_b2, upd_W, upd_b2):
    R = 1000
    grid = (NUM_NODES_IN // R,)
    return pl.pallas_call(
        _tc_dense_body,
        grid=grid,
        in_specs=[
            pl.BlockSpec((NC, R, EDGE_DIM), lambda i: (0, i, 0)),
            pl.BlockSpec((R, NC), lambda i: (i, 0)),
            pl.BlockSpec((EDGE_DIM, HIDDEN_DIM), lambda i: (0, 0)),
            pl.BlockSpec((1, HIDDEN_DIM), lambda i: (0, 0)),
            pl.BlockSpec((HIDDEN_DIM, HIDDEN_DIM), lambda i: (0, 0)),
            pl.BlockSpec((1, HIDDEN_DIM), lambda i: (0, 0)),
        ],
        out_specs=pl.BlockSpec((R, HIDDEN_DIM), lambda i: (i, 0)),
        out_shape=jax.ShapeDtypeStruct((NUM_NODES_IN, HIDDEN_DIM),
                                       jnp.float32),
    )(part, cnt_t, proj_W, proj_b2, upd_W, upd_b2)


@jax.jit
def kernel(edge_attr, edge_index, proj_W, proj_b, upd_W, upd_b):
    dst = edge_index[1].astype(jnp.int32)
    dst2d = dst.reshape(N_ROWS, SUB)
    part, cnt = _sc_segment_sum(edge_attr, dst2d)
    return _tc_dense(
        part,
        cnt.reshape(NC, N_PAD).T,
        proj_W,
        proj_b.reshape(1, HIDDEN_DIM),
        upd_W,
        upd_b.reshape(1, HIDDEN_DIM),
    )
